# baseline (speedup 1.0000x reference)
import functools

import jax
import jax.numpy as jnp
from jax import lax
from jax.experimental import pallas as pl
from jax.experimental.pallas import tpu as pltpu
from jax.experimental.pallas import tpu_sc as plsc

N = 10000
E = 320000
DF = 128
DE = 16
NC = 2
NS = 16
NW = NC * NS
EW = E // NW
C = 80
NCH = EW // C
NG = C // 16
NB = 5
C2 = 40
NCH2 = EW // C2
TT = 4 * N + 16

f32 = jnp.float32
i32 = jnp.int32


def _silu(x):
    return x * jax.nn.sigmoid(x)


def _prep_body(h_ref, wa_ref, wb_ref, a_ref, b_ref):
    h = h_ref[...]
    a_ref[...] = jnp.dot(h, wa_ref[...], preferred_element_type=f32)
    b_ref[...] = jnp.dot(h, wb_ref[...], preferred_element_type=f32)


def _prep(h, wa, wb):
    bn = 1000
    return pl.pallas_call(
        _prep_body,
        grid=(N // bn,),
        in_specs=[
            pl.BlockSpec((bn, DF), lambda i: (i, 0)),
            pl.BlockSpec((DF, DF), lambda i: (0, 0)),
            pl.BlockSpec((DF, DF), lambda i: (0, 0)),
        ],
        out_specs=[
            pl.BlockSpec((bn, DF), lambda i: (i, 0)),
            pl.BlockSpec((bn, DF), lambda i: (i, 0)),
        ],
        out_shape=[
            jax.ShapeDtypeStruct((N, DF), f32),
            jax.ShapeDtypeStruct((N, DF), f32),
        ],
    )(h, wa, wb)


def _gather_body(ap, bp, row2, col2, ctab_hbm, g, cdr, idra, idca, ctab,
                 ab0, ab1, ab2, ab3, ab4, bb0, bb1, bb2, bb3, bb4,
                 cb0, cb1, cb2,
                 semg, semw, semc):
    cid = lax.axis_index("c")
    sid = lax.axis_index("s")
    wid = sid * NC + cid
    base = wid * EW
    pltpu.sync_copy(row2.at[wid], idra)
    pltpu.sync_copy(col2.at[wid], idca)
    pltpu.sync_copy(ctab_hbm, ctab)
    abufs = [ab0, ab1, ab2, ab3, ab4]
    bbufs = [bb0, bb1, bb2, bb3, bb4]
    cbufs = [cb0, cb1, cb2]

    for cb in cbufs:
        def zrow(i, c2, cb=cb):
            cb[pl.ds(i * 16, 16)] = jnp.zeros((16,), f32)
            return c2
        lax.fori_loop(0, C2, zrow, 0)

    lanes = lax.iota(i32, 16)

    def fire_g(j, b):
        pltpu.async_copy(ap.at[idra.at[pl.ds(j * C2, C2)]], abufs[b], semg.at[b])
        pltpu.async_copy(bp.at[idca.at[pl.ds(j * C2, C2)]], bbufs[b], semg.at[b])

    def wait_g(j, b):
        pltpu.make_async_copy(ap.at[idra.at[pl.ds(j * C2, C2)]], abufs[b],
                              semg.at[b]).wait()
        pltpu.make_async_copy(bp.at[idca.at[pl.ds(j * C2, C2)]], bbufs[b],
                              semg.at[b]).wait()

    def fire_w(j, b):
        pltpu.async_copy(abufs[b], g.at[pl.ds(base + j * C2, C2)], semw.at[b])

    def wait_w(j, b):
        pltpu.make_async_copy(abufs[b], g.at[pl.ds(base + j * C2, C2)],
                              semw.at[b]).wait()

    CB = C2 * 16

    def fire_c(j, k):
        pltpu.async_copy(cbufs[k], cdr.at[pl.ds((base + j * C2) * 16, CB)],
                         semc.at[k])

    def wait_c(j, k):
        pltpu.make_async_copy(cbufs[k], cdr.at[pl.ds((base + j * C2) * 16, CB)],
                              semc.at[k]).wait()

    for b in range(3):
        fire_g(b, b)

    def grp(grp_i, b):
        j = grp_i * NB + b

        k = b % 3

        @pl.when(j >= 3)
        def _():
            wait_c(j, k)

        for start in (0, 16, 24):
            ir = idra[pl.ds(j * C2 + start, 16)]
            ic = idca[pl.ds(j * C2 + start, 16)]
            pos = (start + lanes) * 16
            d = []
            for l in range(3):
                cr = plsc.load_gather(ctab, [ir * 4 + l])
                cc = plsc.load_gather(ctab, [ic * 4 + l])
                d.append(cr - cc)
            radial = d[0] * d[0] + d[1] * d[1] + d[2] * d[2]
            for l in range(3):
                plsc.store_scatter(cbufs[k], [pos + l], d[l])
            plsc.store_scatter(cbufs[k], [pos + 3], radial)
        fire_c(j, k)

        wait_g(j, b)

        def addrow(i, c2):
            for l in range(DF // 16):
                sl = pl.ds(l * 16, 16)
                abufs[b][i, sl] = abufs[b][i, sl] + bbufs[b][i, sl]
            return c2

        lax.fori_loop(0, C2, addrow, 0)
        fire_w(j, b)

        @pl.when(j + 3 < NCH2)
        def _():
            b3 = (b + 3) % NB

            @pl.when(j >= 2)
            def _():
                wait_w(j - 2, b3)

            fire_g(j + 3, b3)

    def outer(grp_i, carry):
        for b in range(NB):
            grp(grp_i, b)
        return carry

    lax.fori_loop(0, NCH2 // NB, outer, 0)
    for b in range(NB):
        wait_w(NCH2 - NB + b, b)
    for k in range(3):
        wait_c(NCH2 - 1, k)


_gather = functools.partial(
    pl.kernel,
    out_type=[
        jax.ShapeDtypeStruct((E, DF), f32),
        jax.ShapeDtypeStruct((E * 16,), f32),
    ],
    mesh=plsc.VectorSubcoreMesh(core_axis_name="c", subcore_axis_name="s"),
    scratch_types=[
        pltpu.VMEM((EW,), i32),
        pltpu.VMEM((EW,), i32),
        pltpu.VMEM((TT,), f32),
    ] + [pltpu.VMEM((C2, DF), f32)] * (2 * NB)
      + [pltpu.VMEM((C2 * 16,), f32)] * 3 + [
        pltpu.SemaphoreType.DMA((NB,)),
        pltpu.SemaphoreType.DMA((NB,)),
        pltpu.SemaphoreType.DMA((3,)),
    ],
    compiler_params=pltpu.CompilerParams(needs_layout_passes=False),
)(_gather_body)


def _bdot(x, w):
    return jnp.dot(x.astype(jnp.bfloat16), w.astype(jnp.bfloat16),
                   preferred_element_type=f32)


def _mlp_body(g_ref, cdr_ref, ea_ref, wea_ref, wr_ref, be1_ref, we2_ref, be2_ref,
              wc1_ref, bc1_ref, wc2_ref, ef_ref, tr_ref):
    cdr = cdr_ref[...]
    radial = cdr[:, 3:4]
    m = _silu(g_ref[...] + radial * wr_ref[...] + be1_ref[...]
              + _bdot(ea_ref[...], wea_ref[...]))
    ef = _silu(_bdot(m, we2_ref[...]) + be2_ref[...])
    ef_ref[...] = ef
    cm = _silu(_bdot(ef, wc1_ref[...]) + bc1_ref[...])
    s = jnp.dot(cm, wc2_ref[...], preferred_element_type=f32)
    lane = lax.broadcasted_iota(i32, cdr.shape, 1)
    tr = jnp.where(lane < 3, cdr * s, 0.0) + jnp.where(lane == 3, 1.0, 0.0)
    tr_ref[...] = tr.astype(f32)


def _mlp(g, cdr, ea, wea, wr, be1, we2, be2, wc1, bc1, wc2):
    be = 2000
    full = lambda r, c: pl.BlockSpec((r, c), lambda i: (0, 0))
    return pl.pallas_call(
        _mlp_body,
        grid=(E // be,),
        in_specs=[
            pl.BlockSpec((be, DF), lambda i: (i, 0)),
            pl.BlockSpec((be, 16), lambda i: (i, 0)),
            pl.BlockSpec((be, DE), lambda i: (i, 0)),
            full(DE, DF), full(1, DF), full(1, DF), full(DF, DF), full(1, DF),
            full(DF, DF), full(1, DF), full(DF, 1),
        ],
        out_specs=[
            pl.BlockSpec((be, DF), lambda i: (i, 0)),
            pl.BlockSpec((be, 16), lambda i: (i, 0)),
        ],
        out_shape=[
            jax.ShapeDtypeStruct((E, DF), f32),
            jax.ShapeDtypeStruct((E, 16), f32),
        ],
    )(g, cdr, ea, wea, wr, be1, we2, be2, wc1, bc1, wc2)


def _scatter_body(ef, rowf, aggp, bf0, bf1, bf2, bf3, bf4,
                  ib0, ib1, ib2, ib3, ib4, sef, seml, semi, sems):
    cid = lax.axis_index("c")
    sid = lax.axis_index("s")
    wid = sid * NC + cid
    base = wid * EW
    bufs = [bf0, bf1, bf2, bf3, bf4]
    ibufs = [ib0, ib1, ib2, ib3, ib4]

    def zrow(i, c2):
        for l in range(DF // 16):
            bf0[i, pl.ds(l * 16, 16)] = jnp.zeros((16,), f32)
        return c2

    lax.fori_loop(0, C2, zrow, 0)

    def _zero_slice(roff, nrows):
        nf = nrows // C2
        rem = nrows - nf * C2
        for k in range(nf):
            pltpu.sync_copy(bf0, sef.at[pl.ds(roff + k * C2, C2)])
        if rem:
            pltpu.sync_copy(bf0.at[pl.ds(0, rem)], sef.at[pl.ds(roff + nf * C2, rem)])

    @pl.when(sid < 15)
    def _():
        _zero_slice(sid * 632, 632)

    @pl.when(sid == 15)
    def _():
        _zero_slice(15 * 632, 520)

    plsc.subcore_barrier()

    def fire_l(j, b):
        pltpu.async_copy(ef.at[pl.ds(base + j * C2, C2)], bufs[b], seml.at[b])
        pltpu.async_copy(rowf.at[pl.ds(base + j * C2, C2)], ibufs[b], semi.at[b])

    def wait_l(j, b):
        pltpu.make_async_copy(ef.at[pl.ds(base + j * C2, C2)], bufs[b],
                              seml.at[b]).wait()
        pltpu.make_async_copy(rowf.at[pl.ds(base + j * C2, C2)], ibufs[b],
                              semi.at[b]).wait()

    def fire_s(j, b):
        pltpu.async_copy(bufs[b], sef.at[ibufs[b]], sems.at[b], add=True)

    def wait_s(j, b):
        pltpu.make_async_copy(bufs[b], sef.at[ibufs[b]], sems.at[b]).wait()

    for b in range(3):
        fire_l(b, b)

    def grp(grp_i, b):
        j = grp_i * NB + b
        wait_l(j, b)
        fire_s(j, b)

        @pl.when(j + 3 < NCH2)
        def _():
            b3 = (b + 3) % NB

            @pl.when(j >= 2)
            def _():
                wait_s(j - 2, b3)

            fire_l(j + 3, b3)

    def outer(grp_i, carry):
        for b in range(NB):
            grp(grp_i, b)
        return carry

    lax.fori_loop(0, NCH2 // NB, outer, 0)
    for b in range(NB):
        wait_s(NCH2 - NB + b, b)
    plsc.subcore_barrier()

    @pl.when(sid < 15)
    def _():
        pltpu.sync_copy(sef.at[pl.ds(sid * 632, 632)],
                        aggp.at[pl.ds(cid * N + sid * 632, 632)])

    @pl.when(sid == 15)
    def _():
        pltpu.sync_copy(sef.at[pl.ds(15 * 632, 520)],
                        aggp.at[pl.ds(cid * N + 15 * 632, 520)])


_scatter = functools.partial(
    pl.kernel,
    out_type=jax.ShapeDtypeStruct((2 * N, DF), f32),
    mesh=plsc.VectorSubcoreMesh(core_axis_name="c", subcore_axis_name="s"),
    scratch_types=(
        [pltpu.VMEM((C2, DF), f32)] * NB
        + [pltpu.VMEM((C2,), i32)] * NB
        + [
            pltpu.VMEM_SHARED((N, DF), f32),
            pltpu.SemaphoreType.DMA((NB,)),
            pltpu.SemaphoreType.DMA((NB,)),
            pltpu.SemaphoreType.DMA((NB,)),
        ]
    ),
    compiler_params=pltpu.CompilerParams(needs_layout_passes=False),
)(_scatter_body)


def _transacc_body(tr, row2, tout, idxv, bt0, bt1, bt2, bt3, bt4, ttab, seml):
    cid = lax.axis_index("c")
    sid = lax.axis_index("s")
    wid = sid * NC + cid
    base = wid * EW
    pltpu.sync_copy(row2.at[wid], idxv)
    bufs = [bt0, bt1, bt2, bt3, bt4]

    def ztt(i, c2):
        ttab[pl.ds(i * 16, 16)] = jnp.zeros((16,), f32)
        return c2

    lax.fori_loop(0, TT // 16, ztt, 0)

    lanes = lax.iota(i32, 16)
    m4 = lanes < 4

    def fire_l(j, b):
        pltpu.async_copy(tr.at[pl.ds(base + j * C, C)], bufs[b], seml.at[b])

    def wait_l(j, b):
        pltpu.make_async_copy(tr.at[pl.ds(base + j * C, C)], bufs[b],
                              seml.at[b]).wait()

    for b in range(3):
        fire_l(b, b)

    def grp(grp_i, b):
        j = grp_i * NB + b
        wait_l(j, b)

        def tgrp(gidx, c2):
            for e in range(16):
                r16 = plsc.load_gather(
                    idxv, [jnp.full((16,), j, i32),
                           jnp.full((16,), gidx * 16 + e, i32)])
                pos = r16 * 4 + lanes
                cur = plsc.load_gather(ttab, [pos], mask=m4)
                val = bufs[b][gidx * 16 + e, pl.ds(0, 16)]
                plsc.store_scatter(ttab, [pos], cur + val, mask=m4)
            return c2

        lax.fori_loop(0, NG, tgrp, 0)

        @pl.when(j + 3 < NCH)
        def _():
            fire_l(j + 3, (b + 3) % NB)

    def outer(grp_i, carry):
        for b in range(NB):
            grp(grp_i, b)
        return carry

    lax.fori_loop(0, NCH // NB, outer, 0)
    pltpu.sync_copy(ttab, tout.at[wid])


_transacc = functools.partial(
    pl.kernel,
    out_type=jax.ShapeDtypeStruct((NW, TT), f32),
    mesh=plsc.VectorSubcoreMesh(core_axis_name="c", subcore_axis_name="s"),
    scratch_types=[
        pltpu.VMEM((NCH, C), i32),
    ] + [pltpu.VMEM((C, 16), f32)] * NB + [
        pltpu.VMEM((TT,), f32),
        pltpu.SemaphoreType.DMA((NB,)),
    ],
    compiler_params=pltpu.CompilerParams(needs_layout_passes=False),
)(_transacc_body)


def _node_body(h_ref, cp_ref, a0_ref, a1_ref, tp_ref,
               w1h_ref, w1a_ref, b1_ref, w2_ref, b2_ref, ho_ref, co_ref):
    h = h_ref[...]
    agg = a0_ref[...] + a1_ref[...]
    st = jnp.sum(tp_ref[...], axis=0)
    o = _silu(jnp.dot(h, w1h_ref[...], preferred_element_type=f32)
              + jnp.dot(agg, w1a_ref[...], preferred_element_type=f32)
              + b1_ref[...])
    o = jnp.dot(o, w2_ref[...], preferred_element_type=f32) + b2_ref[...]
    ho_ref[...] = h + o
    cnt = jnp.maximum(st[:, 3:4], 1.0)
    co_ref[...] = cp_ref[...] + st / cnt


def _node(h, coordp, aggp, tpart, w1h, w1a, b1, w2, b2):
    bn = 1000
    nb = N // bn
    full = lambda r, c: pl.BlockSpec((r, c), lambda i: (0, 0))
    return pl.pallas_call(
        _node_body,
        grid=(nb,),
        in_specs=[
            pl.BlockSpec((bn, DF), lambda i: (i, 0)),
            pl.BlockSpec((bn, 4), lambda i: (i, 0)),
            pl.BlockSpec((bn, DF), lambda i: (i, 0)),
            pl.BlockSpec((bn, DF), lambda i: (i + nb, 0)),
            pl.BlockSpec((NW, bn, 4), lambda i: (0, i, 0)),
            full(DF, DF), full(DF, DF), full(1, DF), full(DF, DF), full(1, DF),
        ],
        out_specs=[
            pl.BlockSpec((bn, DF), lambda i: (i, 0)),
            pl.BlockSpec((bn, 4), lambda i: (i, 0)),
        ],
        out_shape=[
            jax.ShapeDtypeStruct((N, DF), f32),
            jax.ShapeDtypeStruct((N, 4), f32),
        ],
    )(h, coordp, aggp, aggp, tpart, w1h, w1a, b1, w2, b2)


def kernel(h, edge_index, coord, edge_attr,
           We1, be1, We2, be2, Wn1, bn1, Wn2, bn2, Wc1, bc1, Wc2):
    row = edge_index[0]
    col = edge_index[1]
    row2 = row.reshape(NW, NCH, C)
    col2 = col.reshape(NW, NCH, C)
    coordp = jnp.pad(coord, ((0, 0), (0, 1)))
    ctab = jnp.pad(coordp.reshape(-1), (0, TT - 4 * N))
    wa = We1[:DF]
    wb = We1[DF:2 * DF]
    wr = We1[2 * DF:2 * DF + 1]
    wea = We1[2 * DF + 1:]

    row2s = row.reshape(NW, EW)
    col2s = col.reshape(NW, EW)
    ap, bp = _prep(h, wa, wb)
    g, cdr_flat = _gather(ap, bp, row2s, col2s, ctab)
    cdr = cdr_flat.reshape(E, 16)
    ef, tr = _mlp(g, cdr, edge_attr, wea, wr, be1.reshape(1, -1), We2,
                  be2.reshape(1, -1), Wc1, bc1.reshape(1, -1), Wc2)
    aggp = _scatter(ef, row)
    tout = _transacc(tr, row2)
    tpart = tout[:, :4 * N].reshape(NW, N, 4)
    ho, co = _node(h, coordp, aggp, tpart, Wn1[:DF], Wn1[DF:],
                   bn1.reshape(1, -1), Wn2, bn2.reshape(1, -1))
    return (ho, co[:, :3], edge_attr, ef)


# speedup vs baseline: 1.1069x; 1.1069x over previous
import functools

import jax
import jax.numpy as jnp
from jax import lax
from jax.experimental import pallas as pl
from jax.experimental.pallas import tpu as pltpu
from jax.experimental.pallas import tpu_sc as plsc

N = 10000
E = 320000
DF = 128
DE = 16
NC = 2
NS = 16
NW = NC * NS
EW = E // NW
C = 80
NCH = EW // C
NG = C // 16
NB = 5
C2 = 40
NCH2 = EW // C2
TT = 4 * N + 16

f32 = jnp.float32
i32 = jnp.int32


def _silu(x):
    return x * jax.nn.sigmoid(x)


def _prep_body(h_ref, wa_ref, wb_ref, a_ref, b_ref):
    h = h_ref[...]
    a_ref[...] = jnp.dot(h, wa_ref[...], preferred_element_type=f32)
    b_ref[...] = jnp.dot(h, wb_ref[...], preferred_element_type=f32)


def _prep(h, wa, wb):
    bn = 1000
    return pl.pallas_call(
        _prep_body,
        grid=(N // bn,),
        in_specs=[
            pl.BlockSpec((bn, DF), lambda i: (i, 0)),
            pl.BlockSpec((DF, DF), lambda i: (0, 0)),
            pl.BlockSpec((DF, DF), lambda i: (0, 0)),
        ],
        out_specs=[
            pl.BlockSpec((bn, DF), lambda i: (i, 0)),
            pl.BlockSpec((bn, DF), lambda i: (i, 0)),
        ],
        out_shape=[
            jax.ShapeDtypeStruct((N, DF), f32),
            jax.ShapeDtypeStruct((N, DF), f32),
        ],
    )(h, wa, wb)


def _coords_body(row2, col2, ctab_hbm, cdr, idra, idca, ctab,
                 cb0, cb1, cb2, cb3, cb4, semw):
    cid = lax.axis_index("c")
    sid = lax.axis_index("s")
    wid = sid * NC + cid
    base = wid * EW
    pltpu.sync_copy(row2.at[wid], idra)
    pltpu.sync_copy(col2.at[wid], idca)
    pltpu.sync_copy(ctab_hbm, ctab)
    cbufs = [cb0, cb1, cb2, cb3, cb4]

    for cb in cbufs:
        def zrow(i, c2, cb=cb):
            cb[i, pl.ds(0, 16)] = jnp.zeros((16,), f32)
            return c2
        lax.fori_loop(0, C, zrow, 0)

    lanes = lax.iota(i32, 16)

    def fire_w(j, b):
        pltpu.async_copy(cbufs[b], cdr.at[pl.ds(base + j * C, C)], semw.at[b])

    def wait_w(j, b):
        pltpu.make_async_copy(cbufs[b], cdr.at[pl.ds(base + j * C, C)],
                              semw.at[b]).wait()

    def grp(grp_i, b):
        j = grp_i * NB + b

        @pl.when(j >= NB)
        def _():
            wait_w(j - NB, b)

        def cgrp(gidx, c2):
            ir = idra[j, pl.ds(gidx * 16, 16)]
            ic = idca[j, pl.ds(gidx * 16, 16)]
            eidx = gidx * 16 + lanes
            d = []
            for l in range(3):
                cr = plsc.load_gather(ctab, [ir * 4 + l])
                cc = plsc.load_gather(ctab, [ic * 4 + l])
                d.append(cr - cc)
            radial = d[0] * d[0] + d[1] * d[1] + d[2] * d[2]
            for l in range(3):
                plsc.store_scatter(cbufs[b], [eidx, jnp.full((16,), l, i32)], d[l])
            plsc.store_scatter(cbufs[b], [eidx, jnp.full((16,), 3, i32)], radial)
            return c2

        lax.fori_loop(0, NG, cgrp, 0)
        fire_w(j, b)

    def outer(grp_i, carry):
        for b in range(NB):
            grp(grp_i, b)
        return carry

    lax.fori_loop(0, NCH // NB, outer, 0)
    for b in range(NB):
        wait_w(NCH - NB + b, b)


_coords = functools.partial(
    pl.kernel,
    out_type=jax.ShapeDtypeStruct((E, 16), f32),
    mesh=plsc.VectorSubcoreMesh(core_axis_name="c", subcore_axis_name="s"),
    scratch_types=[
        pltpu.VMEM((NCH, C), i32),
        pltpu.VMEM((NCH, C), i32),
        pltpu.VMEM((TT,), f32),
    ] + [pltpu.VMEM((C, 16), f32)] * NB + [
        pltpu.SemaphoreType.DMA((NB,)),
    ],
    compiler_params=pltpu.CompilerParams(needs_layout_passes=False),
)(_coords_body)


def _gather_body(ap, bp, row2, col2, g, idra, idca,
                 ab0, ab1, ab2, ab3, ab4, bb0, bb1, bb2, bb3, bb4,
                 semg, semw):
    cid = lax.axis_index("c")
    sid = lax.axis_index("s")
    wid = sid * NC + cid
    base = wid * EW
    pltpu.sync_copy(row2.at[wid], idra)
    pltpu.sync_copy(col2.at[wid], idca)
    abufs = [ab0, ab1, ab2, ab3, ab4]
    bbufs = [bb0, bb1, bb2, bb3, bb4]

    def fire_g(j, b):
        pltpu.async_copy(ap.at[idra.at[j]], abufs[b], semg.at[b])
        pltpu.async_copy(bp.at[idca.at[j]], bbufs[b], semg.at[b])

    def wait_g(j, b):
        pltpu.make_async_copy(ap.at[idra.at[j]], abufs[b], semg.at[b]).wait()
        pltpu.make_async_copy(bp.at[idca.at[j]], bbufs[b], semg.at[b]).wait()

    def fire_w(j, b):
        pltpu.async_copy(abufs[b], g.at[pl.ds(base + j * C2, C2)], semw.at[b])

    def wait_w(j, b):
        pltpu.make_async_copy(abufs[b], g.at[pl.ds(base + j * C2, C2)],
                              semw.at[b]).wait()

    for b in range(3):
        fire_g(b, b)

    def grp(grp_i, b):
        j = grp_i * NB + b
        wait_g(j, b)

        def addrow(i, c2):
            for l in range(DF // 16):
                sl = pl.ds(l * 16, 16)
                abufs[b][i, sl] = abufs[b][i, sl] + bbufs[b][i, sl]
            return c2

        lax.fori_loop(0, C2, addrow, 0)
        fire_w(j, b)

        @pl.when(j + 3 < NCH2)
        def _():
            b3 = (b + 3) % NB

            @pl.when(j >= 2)
            def _():
                wait_w(j - 2, b3)

            fire_g(j + 3, b3)

    def outer(grp_i, carry):
        for b in range(NB):
            grp(grp_i, b)
        return carry

    lax.fori_loop(0, NCH2 // NB, outer, 0)
    for b in range(NB):
        wait_w(NCH2 - NB + b, b)


_gather = functools.partial(
    pl.kernel,
    out_type=jax.ShapeDtypeStruct((E, DF), f32),
    mesh=plsc.VectorSubcoreMesh(core_axis_name="c", subcore_axis_name="s"),
    scratch_types=[
        pltpu.VMEM((NCH2, C2), i32),
        pltpu.VMEM((NCH2, C2), i32),
    ] + [pltpu.VMEM((C2, DF), f32)] * (2 * NB) + [
        pltpu.SemaphoreType.DMA((NB,)),
        pltpu.SemaphoreType.DMA((NB,)),
    ],
    compiler_params=pltpu.CompilerParams(needs_layout_passes=False),
)(_gather_body)


def _bdot(x, w):
    return jnp.dot(x.astype(jnp.bfloat16), w.astype(jnp.bfloat16),
                   preferred_element_type=f32)


def _mlp_body(g_ref, cdr_ref, ea_ref, wea_ref, wr_ref, be1_ref, we2_ref, be2_ref,
              wc1_ref, bc1_ref, wc2_ref, ef_ref, tr_ref):
    cdr = cdr_ref[...]
    radial = cdr[:, 3:4]
    m = _silu(g_ref[...] + radial * wr_ref[...] + be1_ref[...]
              + _bdot(ea_ref[...], wea_ref[...]))
    ef = _silu(_bdot(m, we2_ref[...]) + be2_ref[...])
    ef_ref[...] = ef
    cm = _silu(_bdot(ef, wc1_ref[...]) + bc1_ref[...])
    s = jnp.dot(cm, wc2_ref[...], preferred_element_type=f32)
    lane = lax.broadcasted_iota(i32, cdr.shape, 1)
    tr = jnp.where(lane < 3, cdr * s, 0.0) + jnp.where(lane == 3, 1.0, 0.0)
    tr_ref[...] = tr.astype(f32)


def _mlp(g, cdr, ea, wea, wr, be1, we2, be2, wc1, bc1, wc2):
    be = 4000
    full = lambda r, c: pl.BlockSpec((r, c), lambda i: (0, 0))
    return pl.pallas_call(
        _mlp_body,
        grid=(E // be,),
        in_specs=[
            pl.BlockSpec((be, DF), lambda i: (i, 0)),
            pl.BlockSpec((be, 16), lambda i: (i, 0)),
            pl.BlockSpec((be, DE), lambda i: (i, 0)),
            full(DE, DF), full(1, DF), full(1, DF), full(DF, DF), full(1, DF),
            full(DF, DF), full(1, DF), full(DF, 1),
        ],
        out_specs=[
            pl.BlockSpec((be, DF), lambda i: (i, 0)),
            pl.BlockSpec((be, 16), lambda i: (i, 0)),
        ],
        out_shape=[
            jax.ShapeDtypeStruct((E, DF), f32),
            jax.ShapeDtypeStruct((E, 16), f32),
        ],
    )(g, cdr, ea, wea, wr, be1, we2, be2, wc1, bc1, wc2)


def _scatter_body(ef, rowf, aggp, bf0, bf1, bf2, bf3, bf4,
                  ib0, ib1, ib2, ib3, ib4, sef, seml, semi, sems):
    cid = lax.axis_index("c")
    sid = lax.axis_index("s")
    wid = sid * NC + cid
    base = wid * EW
    bufs = [bf0, bf1, bf2, bf3, bf4]
    ibufs = [ib0, ib1, ib2, ib3, ib4]

    def zrow(i, c2):
        for l in range(DF // 16):
            bf0[i, pl.ds(l * 16, 16)] = jnp.zeros((16,), f32)
        return c2

    lax.fori_loop(0, C2, zrow, 0)

    def _zero_slice(roff, nrows):
        nf = nrows // C2
        rem = nrows - nf * C2
        for k in range(nf):
            pltpu.sync_copy(bf0, sef.at[pl.ds(roff + k * C2, C2)])
        if rem:
            pltpu.sync_copy(bf0.at[pl.ds(0, rem)], sef.at[pl.ds(roff + nf * C2, rem)])

    @pl.when(sid < 15)
    def _():
        _zero_slice(sid * 632, 632)

    @pl.when(sid == 15)
    def _():
        _zero_slice(15 * 632, 520)

    plsc.subcore_barrier()

    def fire_l(j, b):
        pltpu.async_copy(ef.at[pl.ds(base + j * C2, C2)], bufs[b], seml.at[b])
        pltpu.async_copy(rowf.at[pl.ds(base + j * C2, C2)], ibufs[b], semi.at[b])

    def wait_l(j, b):
        pltpu.make_async_copy(ef.at[pl.ds(base + j * C2, C2)], bufs[b],
                              seml.at[b]).wait()
        pltpu.make_async_copy(rowf.at[pl.ds(base + j * C2, C2)], ibufs[b],
                              semi.at[b]).wait()

    def fire_s(j, b):
        pltpu.async_copy(bufs[b], sef.at[ibufs[b]], sems.at[b], add=True)

    def wait_s(j, b):
        pltpu.make_async_copy(bufs[b], sef.at[ibufs[b]], sems.at[b]).wait()

    for b in range(3):
        fire_l(b, b)

    def grp(grp_i, b):
        j = grp_i * NB + b
        wait_l(j, b)
        fire_s(j, b)

        @pl.when(j + 3 < NCH2)
        def _():
            b3 = (b + 3) % NB

            @pl.when(j >= 2)
            def _():
                wait_s(j - 2, b3)

            fire_l(j + 3, b3)

    def outer(grp_i, carry):
        for b in range(NB):
            grp(grp_i, b)
        return carry

    lax.fori_loop(0, NCH2 // NB, outer, 0)
    for b in range(NB):
        wait_s(NCH2 - NB + b, b)
    plsc.subcore_barrier()

    @pl.when(sid < 15)
    def _():
        pltpu.sync_copy(sef.at[pl.ds(sid * 632, 632)],
                        aggp.at[pl.ds(cid * N + sid * 632, 632)])

    @pl.when(sid == 15)
    def _():
        pltpu.sync_copy(sef.at[pl.ds(15 * 632, 520)],
                        aggp.at[pl.ds(cid * N + 15 * 632, 520)])


_scatter = functools.partial(
    pl.kernel,
    out_type=jax.ShapeDtypeStruct((2 * N, DF), f32),
    mesh=plsc.VectorSubcoreMesh(core_axis_name="c", subcore_axis_name="s"),
    scratch_types=(
        [pltpu.VMEM((C2, DF), f32)] * NB
        + [pltpu.VMEM((C2,), i32)] * NB
        + [
            pltpu.VMEM_SHARED((N, DF), f32),
            pltpu.SemaphoreType.DMA((NB,)),
            pltpu.SemaphoreType.DMA((NB,)),
            pltpu.SemaphoreType.DMA((NB,)),
        ]
    ),
    compiler_params=pltpu.CompilerParams(needs_layout_passes=False),
)(_scatter_body)


def _transacc_body(tr, row2, tout, idxv, bt0, bt1, bt2, bt3, bt4, ttab, seml):
    cid = lax.axis_index("c")
    sid = lax.axis_index("s")
    wid = sid * NC + cid
    base = wid * EW
    pltpu.sync_copy(row2.at[wid], idxv)
    bufs = [bt0, bt1, bt2, bt3, bt4]

    def ztt(i, c2):
        ttab[pl.ds(i * 16, 16)] = jnp.zeros((16,), f32)
        return c2

    lax.fori_loop(0, TT // 16, ztt, 0)

    lanes = lax.iota(i32, 16)
    m4 = lanes < 4

    def fire_l(j, b):
        pltpu.async_copy(tr.at[pl.ds(base + j * C, C)], bufs[b], seml.at[b])

    def wait_l(j, b):
        pltpu.make_async_copy(tr.at[pl.ds(base + j * C, C)], bufs[b],
                              seml.at[b]).wait()

    for b in range(3):
        fire_l(b, b)

    def grp(grp_i, b):
        j = grp_i * NB + b
        wait_l(j, b)

        def tgrp(gidx, c2):
            for e in range(16):
                r16 = plsc.load_gather(
                    idxv, [jnp.full((16,), j, i32),
                           jnp.full((16,), gidx * 16 + e, i32)])
                pos = r16 * 4 + lanes
                cur = plsc.load_gather(ttab, [pos], mask=m4)
                val = bufs[b][gidx * 16 + e, pl.ds(0, 16)]
                plsc.store_scatter(ttab, [pos], cur + val, mask=m4)
            return c2

        lax.fori_loop(0, NG, tgrp, 0)

        @pl.when(j + 3 < NCH)
        def _():
            fire_l(j + 3, (b + 3) % NB)

    def outer(grp_i, carry):
        for b in range(NB):
            grp(grp_i, b)
        return carry

    lax.fori_loop(0, NCH // NB, outer, 0)
    pltpu.sync_copy(ttab, tout.at[wid])


_transacc = functools.partial(
    pl.kernel,
    out_type=jax.ShapeDtypeStruct((NW, TT), f32),
    mesh=plsc.VectorSubcoreMesh(core_axis_name="c", subcore_axis_name="s"),
    scratch_types=[
        pltpu.VMEM((NCH, C), i32),
    ] + [pltpu.VMEM((C, 16), f32)] * NB + [
        pltpu.VMEM((TT,), f32),
        pltpu.SemaphoreType.DMA((NB,)),
    ],
    compiler_params=pltpu.CompilerParams(needs_layout_passes=False),
)(_transacc_body)


def _node_body(h_ref, cp_ref, a0_ref, a1_ref, tp_ref,
               w1h_ref, w1a_ref, b1_ref, w2_ref, b2_ref, ho_ref, co_ref):
    h = h_ref[...]
    agg = a0_ref[...] + a1_ref[...]
    st = jnp.sum(tp_ref[...], axis=0)
    o = _silu(jnp.dot(h, w1h_ref[...], preferred_element_type=f32)
              + jnp.dot(agg, w1a_ref[...], preferred_element_type=f32)
              + b1_ref[...])
    o = jnp.dot(o, w2_ref[...], preferred_element_type=f32) + b2_ref[...]
    ho_ref[...] = h + o
    cnt = jnp.maximum(st[:, 3:4], 1.0)
    co_ref[...] = cp_ref[...] + st / cnt


def _node(h, coordp, aggp, tpart, w1h, w1a, b1, w2, b2):
    bn = 1000
    nb = N // bn
    full = lambda r, c: pl.BlockSpec((r, c), lambda i: (0, 0))
    return pl.pallas_call(
        _node_body,
        grid=(nb,),
        in_specs=[
            pl.BlockSpec((bn, DF), lambda i: (i, 0)),
            pl.BlockSpec((bn, 4), lambda i: (i, 0)),
            pl.BlockSpec((bn, DF), lambda i: (i, 0)),
            pl.BlockSpec((bn, DF), lambda i: (i + nb, 0)),
            pl.BlockSpec((NW, bn, 4), lambda i: (0, i, 0)),
            full(DF, DF), full(DF, DF), full(1, DF), full(DF, DF), full(1, DF),
        ],
        out_specs=[
            pl.BlockSpec((bn, DF), lambda i: (i, 0)),
            pl.BlockSpec((bn, 4), lambda i: (i, 0)),
        ],
        out_shape=[
            jax.ShapeDtypeStruct((N, DF), f32),
            jax.ShapeDtypeStruct((N, 4), f32),
        ],
    )(h, coordp, aggp, aggp, tpart, w1h, w1a, b1, w2, b2)


def kernel(h, edge_index, coord, edge_attr,
           We1, be1, We2, be2, Wn1, bn1, Wn2, bn2, Wc1, bc1, Wc2):
    row = edge_index[0]
    col = edge_index[1]
    row2 = row.reshape(NW, NCH, C)
    col2 = col.reshape(NW, NCH, C)
    coordp = jnp.pad(coord, ((0, 0), (0, 1)))
    ctab = jnp.pad(coordp.reshape(-1), (0, TT - 4 * N))
    wa = We1[:DF]
    wb = We1[DF:2 * DF]
    wr = We1[2 * DF:2 * DF + 1]
    wea = We1[2 * DF + 1:]

    row2s = row.reshape(NW, NCH2, C2)
    col2s = col.reshape(NW, NCH2, C2)
    cdr = _coords(row2, col2, ctab)
    ap, bp = _prep(h, wa, wb)
    g = _gather(ap, bp, row2s, col2s)
    ef, tr = _mlp(g, cdr, edge_attr, wea, wr, be1.reshape(1, -1), We2,
                  be2.reshape(1, -1), Wc1, bc1.reshape(1, -1), Wc2)
    aggp = _scatter(ef, row)
    tout = _transacc(tr, row2)
    tpart = tout[:, :4 * N].reshape(NW, N, 4)
    ho, co = _node(h, coordp, aggp, tpart, Wn1[:DF], Wn1[DF:],
                   bn1.reshape(1, -1), Wn2, bn2.reshape(1, -1))
    return (ho, co[:, :3], edge_attr, ef)
